# baseline (speedup 1.0000x reference)
import functools

import jax
import jax.numpy as jnp
from jax import lax
from jax.experimental import pallas as pl
from jax.experimental.pallas import tpu as pltpu
from jax.experimental.pallas import tpu_sc as plsc

VOCAB = 1000000
DIM = 64
PAD = 0

_NC = 2
_NS = 16
_L = 16
_NW = _NC * _NS

_B = 8192
_BPW = _B // _NW
_IDX_MINOR = 128
_ROWS_PER_W = _BPW // _IDX_MINOR


def _sc_body(idx_hbm, table_hbm, out_hbm, idx_v, rows_v, sem):
    wid = lax.axis_index("s") * _NC + lax.axis_index("c")
    base = wid * _BPW

    for j in range(_ROWS_PER_W):
        pltpu.sync_copy(
            idx_hbm.at[_ROWS_PER_W * wid + j],
            idx_v.at[pl.ds(j * _IDX_MINOR, _IDX_MINOR)],
        )

    for t in range(_BPW // _L):
        ivec = idx_v[pl.ds(t * _L, _L)]
        for r in range(_L):
            iv = ivec[r]
            pltpu.make_async_copy(
                table_hbm.at[pl.ds(iv, 1), :],
                rows_v.at[pl.ds(t * _L + r, 1), :],
                sem,
            ).start()
    pltpu.make_async_copy(
        table_hbm.at[pl.ds(0, _BPW), :], rows_v, sem
    ).wait()

    ones = jnp.ones((_L,), jnp.int32)
    zeros = jnp.zeros((_L,), jnp.int32)
    acc = zeros
    for t in range(_BPW // _L):
        iv = idx_v[pl.ds(t * _L, _L)]
        acc = acc | jnp.where(iv == PAD, ones, zeros)
    has_pad = jnp.max(acc)

    @pl.when(has_pad > 0)
    def _zero_pad_rows():
        onesf = jnp.ones((_L,), jnp.float32)
        zerosf = jnp.zeros((_L,), jnp.float32)
        dnums = lax.GatherDimensionNumbers(
            offset_dims=(), collapsed_slice_dims=(0,), start_index_map=(0,))
        for t in range(_BPW // _L):
            iv = idx_v[pl.ds(t * _L, _L)]
            scale = jnp.where(iv == PAD, zerosf, onesf)
            for r in range(_L):
                row = t * _L + r
                bidx = jnp.full((_L, 1), r, jnp.int32)
                bvec = lax.gather(
                    scale, bidx, dnums, (1,),
                    mode=lax.GatherScatterMode.PROMISE_IN_BOUNDS)
                for cchunk in range(DIM // _L):
                    sl = pl.ds(cchunk * _L, _L)
                    rows_v[row, sl] = rows_v[row, sl] * bvec

    pltpu.sync_copy(rows_v, out_hbm.at[pl.ds(base, _BPW)])


def _lookup(idx2d, word_lut):
    mesh = plsc.VectorSubcoreMesh(core_axis_name="c", subcore_axis_name="s")
    return pl.kernel(
        _sc_body,
        out_type=jax.ShapeDtypeStruct((_B, DIM), jnp.float32),
        mesh=mesh,
        compiler_params=pltpu.CompilerParams(needs_layout_passes=False),
        scratch_types=[
            pltpu.VMEM((_BPW,), jnp.int32),
            pltpu.VMEM((_BPW, DIM), jnp.float32),
            pltpu.SemaphoreType.DMA,
        ],
    )(idx2d, word_lut)


def kernel(src_input, word_lut):
    seq, batch, _ = src_input.shape
    idx2d = src_input[:, :, 0].reshape(_B // _IDX_MINOR, _IDX_MINOR)
    out = _lookup(idx2d, word_lut)
    return out.reshape(seq, batch, DIM)
